# baseline (speedup 1.0000x reference)
import jax
import jax.numpy as jnp
from jax.experimental import pallas as pl
from jax.experimental.pallas import tpu as pltpu

T = 48
C = 34
WAVE = 4
HIDDEN = 64
N_EMBED = 256
BLOCKS = 4
PATCH_DIM = WAVE * C
N_TOK = T // WAVE


def _fused_body(vl_ref, x_ref, win_ref, bin_ref, w1_ref, b1_ref, w2_ref,
                b2_ref, cb_ref, out_ref):
    vl = vl_ref[0, 0]
    x = x_ref[...]
    row = jax.lax.broadcasted_iota(jnp.int32, (N_TOK, PATCH_DIM), 0)
    col = jax.lax.broadcasted_iota(jnp.int32, (N_TOK, PATCH_DIM), 1)
    t = row * WAVE + col // C
    x = jnp.where(t < vl, x, 0.0)

    z = jnp.dot(x, win_ref[...], preferred_element_type=jnp.float32)
    z = z + bin_ref[...]

    for i in range(BLOCKS):
        mu = z.mean(axis=-1, keepdims=True)
        var = ((z - mu) ** 2).mean(axis=-1, keepdims=True)
        h = (z - mu) / jnp.sqrt(var + 1e-5)
        h = jnp.dot(h, w1_ref[i], preferred_element_type=jnp.float32) + b1_ref[i][None, :]
        h = jax.nn.gelu(h)
        h = jnp.dot(h, w2_ref[i], preferred_element_type=jnp.float32) + b2_ref[i][None, :]
        z = z + h

    cb = cb_ref[...]
    z2 = jnp.sum(z * z, axis=-1, keepdims=True)
    zc = jax.lax.dot_general(z, cb, (((1,), (1,)), ((), ())),
                             preferred_element_type=jnp.float32)
    c2 = jnp.sum(cb * cb, axis=-1)
    d = z2 - 2.0 * zc + c2[None, :]

    m = jnp.min(d, axis=-1, keepdims=True)
    idx = jax.lax.broadcasted_iota(jnp.int32, (N_TOK, N_EMBED), 1)
    ids = jnp.min(jnp.where(d == m, idx, N_EMBED), axis=-1)
    out_ref[...] = jnp.broadcast_to(ids[:, None], (N_TOK, 128))


def kernel(ts, W_in, b_in, blocks_W1, blocks_b1, blocks_W2, blocks_b2,
           codebook, W_out, b_out, valid_len):
    patches = ts.reshape(N_TOK, PATCH_DIM)
    vl = jnp.asarray(valid_len, jnp.int32).reshape(1, 1)
    out = pl.pallas_call(
        _fused_body,
        out_shape=jax.ShapeDtypeStruct((N_TOK, 128), jnp.int32),
        in_specs=[
            pl.BlockSpec(memory_space=pltpu.SMEM),
            pl.BlockSpec(memory_space=pltpu.VMEM),
            pl.BlockSpec(memory_space=pltpu.VMEM),
            pl.BlockSpec(memory_space=pltpu.VMEM),
            pl.BlockSpec(memory_space=pltpu.VMEM),
            pl.BlockSpec(memory_space=pltpu.VMEM),
            pl.BlockSpec(memory_space=pltpu.VMEM),
            pl.BlockSpec(memory_space=pltpu.VMEM),
            pl.BlockSpec(memory_space=pltpu.VMEM),
        ],
        out_specs=pl.BlockSpec(memory_space=pltpu.VMEM),
    )(vl, patches, W_in, b_in.reshape(1, HIDDEN), blocks_W1, blocks_b1,
      blocks_W2, blocks_b2, codebook)
    return out[:, 0].reshape(1, N_TOK)
